# baseline (speedup 1.0000x reference)
import functools

import jax
import jax.numpy as jnp
import numpy as np
from jax import lax
from jax.experimental import pallas as pl
from jax.experimental.pallas import tpu as pltpu
from jax.experimental.pallas import tpu_sc as plsc

WIDTH = 1024
HEIGHT = 1024
NBATCH = 16
NPTS = 42
TOTAL = NBATCH * NPTS
LANES = 16
NCHUNK = TOTAL // LANES
NWORK = 14
NSLOTS = 3

_BOFFS = (np.arange(TOTAL, dtype=np.int32) // NPTS) * WIDTH

_mesh = plsc.VectorSubcoreMesh(core_axis_name="c", subcore_axis_name="s")


@functools.partial(
    pl.kernel,
    out_type=jax.ShapeDtypeStruct((LANES,), jnp.float32),
    mesh=_mesh,
    scratch_types=[
        pltpu.VMEM((2 * TOTAL,), jnp.float32),
        pltpu.VMEM((TOTAL,), jnp.int32),
        pltpu.VMEM((NSLOTS * LANES, LANES), jnp.float32),
        pltpu.VMEM((NWORK, LANES), jnp.float32),
        pltpu.VMEM((LANES,), jnp.float32),
        pltpu.HBM((NWORK, LANES), jnp.float32),
        pltpu.SemaphoreType.DMA,
    ],
)
def _mask_loss_sc(pred_hbm, boffs_hbm, dist_hbm, out_hbm,
                  pred_v, boffs_v, seg_v, part_v, out_v, parts_hbm, sem):
    cid = lax.axis_index("c")
    sid = lax.axis_index("s")

    @pl.when(jnp.logical_and(cid == 0, sid < NWORK))
    def _():
        pltpu.sync_copy(pred_hbm, pred_v)
        pltpu.sync_copy(boffs_hbm, boffs_v)
        lane = lax.iota(jnp.int32, LANES)
        zero = jnp.zeros((LANES,), jnp.float32)

        copies, ts, iis = [], [], []
        for l in range(NSLOTS):
            c = sid * NSLOTS + l
            boffs = boffs_v[pl.ds(c * LANES, LANES)]
            for h in range(2):
                v = pred_v[pl.ds(c * 2 * LANES + h * LANES, LANES)]
                ii = jnp.minimum(
                    ((v + 1.0) * (WIDTH * 0.5)).astype(jnp.int32), WIDTH - 1
                )
                q = v / v
                t = jnp.where(q >= 1.0, jnp.float32(1.0), 0.0 * q)
                ts.append(t)
                iis.append(ii)
                for j in range(LANES // 2):
                    p_local = l * LANES + h * (LANES // 2) + j
                    row = boffs[h * (LANES // 2) + j] + ii[2 * j]
                    yj = ii[2 * j + 1]
                    ybase = pl.multiple_of(
                        jnp.bitwise_and(yj, jnp.int32(-16)), 16
                    )
                    cp = pltpu.make_async_copy(
                        dist_hbm.at[row, pl.ds(ybase, LANES)],
                        seg_v.at[p_local],
                        sem,
                    )
                    cp.start()
                    copies.append(cp)

        for cp in copies:
            cp.wait()
        acc = zero
        for l in range(NSLOTS):
            for h in range(2):
                t = ts[2 * l + h]
                ii = iis[2 * l + h]
                for j in range(LANES // 2):
                    p_local = l * LANES + h * (LANES // 2) + j
                    w = t[2 * j] * t[2 * j + 1]
                    d = jnp.bitwise_and(ii[2 * j + 1], jnp.int32(15))
                    seg = seg_v[p_local]
                    sel = lane == jnp.full((LANES,), d, jnp.int32)
                    acc = acc + jnp.where(
                        sel, seg * jnp.full((LANES,), w, jnp.float32), zero
                    )

        out_v[...] = acc
        pltpu.sync_copy(out_v, parts_hbm.at[sid])

    @pl.when(cid == 0)
    def _():
        plsc.subcore_barrier()

        @pl.when(sid == 0)
        def _():
            pltpu.sync_copy(parts_hbm, part_v)
            tot_v = jnp.zeros((LANES,), jnp.float32)
            for t in range(NWORK):
                tot_v = tot_v + part_v[t]
            total = tot_v[0]
            for i in range(1, LANES):
                total = total + tot_v[i]
            out_v[...] = jnp.full((LANES,), total * (1.0 / TOTAL), jnp.float32)
            pltpu.sync_copy(out_v, out_hbm)


def kernel(predict, distmap):
    out = _mask_loss_sc(predict.reshape(-1), jnp.asarray(_BOFFS),
                        distmap.reshape(NBATCH * WIDTH, HEIGHT))
    return out[0]
